# baseline (speedup 1.0000x reference)
import jax
import jax.numpy as jnp
from jax import lax
from jax.experimental import pallas as pl
from jax.experimental.pallas import tpu as pltpu

_S = 4
_M = 4
_K = 256
_D = 32
_DT = _S * _D
_KT = _S * _K


def _pack_kernel(C_ref, Wc_ref, Wx_ref, b_ref,
                 candT_ref, cand_ref, cnorm_ref, wx_ref, bf_ref):
    candT_ref[...] = jnp.zeros_like(candT_ref)
    wx_ref[...] = jnp.zeros_like(wx_ref)
    for m in range(_M):
        for s in range(_S):
            cm = jnp.dot(C_ref[s, m].astype(jnp.bfloat16),
                         Wc_ref[s, m].astype(jnp.bfloat16),
                         preferred_element_type=jnp.float32)
            candT_ref[m, s * _D:(s + 1) * _D, s * _K:(s + 1) * _K] = (
                (cm.T.astype(jnp.bfloat16) * jnp.bfloat16(2.0)))
            hi = cm.astype(jnp.bfloat16)
            mid = (cm - hi.astype(jnp.float32)).astype(jnp.bfloat16)
            lo = ((cm - hi.astype(jnp.float32)) - mid.astype(jnp.float32)
                  ).astype(jnp.bfloat16)
            cand_ref[m, s, :, 0 * _D:1 * _D] = hi
            cand_ref[m, s, :, 1 * _D:2 * _D] = mid
            cand_ref[m, s, :, 2 * _D:3 * _D] = lo
            cnorm_ref[m, :, s * _K:(s + 1) * _K] = jnp.sum(cm * cm, axis=-1)[None, :]
            wx_ref[m, s * _D:(s + 1) * _D, s * _D:(s + 1) * _D] = (
                Wx_ref[s, m].astype(jnp.bfloat16))
            bf_ref[m, :, s * _D:(s + 1) * _D] = b_ref[s, m][None, :]


def _encode_kernel(x_ref, candT_ref, cand_ref, cnorm_ref, wx_ref, bf_ref,
                   codes_ref, xhat_ref):
    xs = x_ref[...]
    bblk = xs.shape[0]
    xhat = jnp.zeros_like(xs)
    codes_by_sm = [[None] * _M for _ in range(_S)]
    iota = lax.broadcasted_iota(jnp.int32, (bblk, _K), 1)
    for m in range(_M):
        if m == 0:
            cond = jnp.broadcast_to(bf_ref[0], (bblk, _DT))
        else:
            cond = jnp.dot(xhat.astype(jnp.bfloat16), wx_ref[m],
                           preferred_element_type=jnp.float32) + bf_ref[m]
        r2 = (xs - xhat) - cond
        two_dot = jnp.dot(r2.astype(jnp.bfloat16), candT_ref[m],
                          preferred_element_type=jnp.float32)
        sel_parts = []
        for s in range(_S):
            rs = r2[:, s * _D:(s + 1) * _D]
            a = jnp.sum(rs * rs, axis=-1, keepdims=True)
            ds = (a - two_dot[:, s * _K:(s + 1) * _K]) + cnorm_ref[m][:, s * _K:(s + 1) * _K]
            code = jnp.argmin(ds, axis=-1).astype(jnp.int32)
            codes_by_sm[s][m] = code
            oh = (iota == code[:, None]).astype(jnp.bfloat16)
            g = jnp.dot(oh, cand_ref[m, s],
                        preferred_element_type=jnp.float32)
            sel_parts.append((g[:, 0 * _D:1 * _D] + g[:, 1 * _D:2 * _D])
                             + g[:, 2 * _D:3 * _D])
        sel = jnp.concatenate(sel_parts, axis=-1)
        xhat = xhat + (sel + cond)
    xhat_ref[...] = xhat
    flat = [codes_by_sm[s][m] for s in range(_S) for m in range(_M)]
    codes_ref[...] = jnp.stack(flat, axis=-1)


def kernel(x, C, Wc, Wx, b):
    B = x.shape[0]
    f32 = jnp.float32
    bf16 = jnp.bfloat16

    candT, cand, cnorm, wxbd, bf = pl.pallas_call(
        _pack_kernel,
        out_shape=[
            jax.ShapeDtypeStruct((_M, _DT, _KT), bf16),
            jax.ShapeDtypeStruct((_M, _S, _K, 3 * _D), bf16),
            jax.ShapeDtypeStruct((_M, 1, _KT), f32),
            jax.ShapeDtypeStruct((_M, _DT, _DT), bf16),
            jax.ShapeDtypeStruct((_M, 1, _DT), f32),
        ],
    )(C, Wc, Wx, b)

    bblk = 2048 if B % 2048 == 0 else B
    nb = B // bblk

    codes, xhat = pl.pallas_call(
        _encode_kernel,
        grid=(nb,),
        in_specs=[
            pl.BlockSpec((bblk, _DT), lambda i: (i, 0)),
            pl.BlockSpec((_M, _DT, _KT), lambda i: (0, 0, 0)),
            pl.BlockSpec((_M, _S, _K, 3 * _D), lambda i: (0, 0, 0, 0)),
            pl.BlockSpec((_M, 1, _KT), lambda i: (0, 0, 0)),
            pl.BlockSpec((_M, _DT, _DT), lambda i: (0, 0, 0)),
            pl.BlockSpec((_M, 1, _DT), lambda i: (0, 0, 0)),
        ],
        out_specs=[
            pl.BlockSpec((bblk, _S * _M), lambda i: (i, 0)),
            pl.BlockSpec((bblk, _DT), lambda i: (i, 0)),
        ],
        out_shape=[
            jax.ShapeDtypeStruct((B, _S * _M), jnp.int32),
            jax.ShapeDtypeStruct((B, _DT), f32),
        ],
        compiler_params=pltpu.CompilerParams(
            dimension_semantics=("parallel",),
        ),
    )(x, candT, cand, cnorm, wxbd, bf)
    return codes, xhat


# speedup vs baseline: 1.3019x; 1.3019x over previous
import jax
import jax.numpy as jnp
from jax import lax
from jax.experimental import pallas as pl
from jax.experimental.pallas import tpu as pltpu

_S = 4
_M = 4
_K = 256
_D = 32
_DT = _S * _D
_KT = _S * _K


def _pack_kernel(C_ref, Wc_ref, Wx_ref, b_ref,
                 candT_ref, cand_ref, cnorm_ref, wx_ref, bf_ref):
    candT_ref[...] = jnp.zeros_like(candT_ref)
    wx_ref[...] = jnp.zeros_like(wx_ref)
    for m in range(_M):
        for s in range(_S):
            cm = jnp.dot(C_ref[s, m].astype(jnp.bfloat16),
                         Wc_ref[s, m].astype(jnp.bfloat16),
                         preferred_element_type=jnp.float32)
            candT_ref[m, s * _D:(s + 1) * _D, s * _K:(s + 1) * _K] = (
                (cm.T.astype(jnp.bfloat16) * jnp.bfloat16(2.0)))
            hi = cm.astype(jnp.bfloat16)
            mid = (cm - hi.astype(jnp.float32)).astype(jnp.bfloat16)
            lo = ((cm - hi.astype(jnp.float32)) - mid.astype(jnp.float32)
                  ).astype(jnp.bfloat16)
            cand_ref[m, s, :, 0 * _D:1 * _D] = hi
            cand_ref[m, s, :, 1 * _D:2 * _D] = mid
            cand_ref[m, s, :, 2 * _D:3 * _D] = lo
            cnorm_ref[m, :, s * _K:(s + 1) * _K] = jnp.sum(cm * cm, axis=-1)[None, :]
            wx_ref[m, s * _D:(s + 1) * _D, s * _D:(s + 1) * _D] = (
                Wx_ref[s, m].astype(jnp.bfloat16))
            bf_ref[m, :, s * _D:(s + 1) * _D] = b_ref[s, m][None, :]


def _encode_kernel(x_ref, candT_ref, cand_ref, cnorm_ref, wx_ref, bf_ref,
                   codes_ref, xhat_ref):
    xs = x_ref[...]
    bblk = xs.shape[0]
    xhat = jnp.zeros_like(xs)
    codes_by_sm = [[None] * _M for _ in range(_S)]
    iota = lax.broadcasted_iota(jnp.int32, (bblk, _K), 1)
    for m in range(_M):
        if m == 0:
            cond = jnp.broadcast_to(bf_ref[0], (bblk, _DT))
        else:
            cond = jnp.dot(xhat.astype(jnp.bfloat16), wx_ref[m],
                           preferred_element_type=jnp.float32) + bf_ref[m]
        r2 = (xs - xhat) - cond
        two_dot = jnp.dot(r2.astype(jnp.bfloat16), candT_ref[m],
                          preferred_element_type=jnp.float32)
        sel_parts = []
        for s in range(_S):
            ds = cnorm_ref[m][:, s * _K:(s + 1) * _K] - two_dot[:, s * _K:(s + 1) * _K]
            code = jnp.argmin(ds, axis=-1).astype(jnp.int32)
            codes_by_sm[s][m] = code
            oh = (iota == code[:, None]).astype(jnp.bfloat16)
            g = jnp.dot(oh, cand_ref[m, s],
                        preferred_element_type=jnp.float32)
            sel_parts.append((g[:, 0 * _D:1 * _D] + g[:, 1 * _D:2 * _D])
                             + g[:, 2 * _D:3 * _D])
        sel = jnp.concatenate(sel_parts, axis=-1)
        xhat = xhat + (sel + cond)
    xhat_ref[...] = xhat
    flat = [codes_by_sm[s][m] for s in range(_S) for m in range(_M)]
    codes_ref[...] = jnp.stack(flat, axis=-1)


def kernel(x, C, Wc, Wx, b):
    B = x.shape[0]
    f32 = jnp.float32
    bf16 = jnp.bfloat16

    candT, cand, cnorm, wxbd, bf = pl.pallas_call(
        _pack_kernel,
        out_shape=[
            jax.ShapeDtypeStruct((_M, _DT, _KT), bf16),
            jax.ShapeDtypeStruct((_M, _S, _K, 3 * _D), bf16),
            jax.ShapeDtypeStruct((_M, 1, _KT), f32),
            jax.ShapeDtypeStruct((_M, _DT, _DT), bf16),
            jax.ShapeDtypeStruct((_M, 1, _DT), f32),
        ],
    )(C, Wc, Wx, b)

    bblk = 2048 if B % 2048 == 0 else B
    nb = B // bblk

    codes, xhat = pl.pallas_call(
        _encode_kernel,
        grid=(nb,),
        in_specs=[
            pl.BlockSpec((bblk, _DT), lambda i: (i, 0)),
            pl.BlockSpec((_M, _DT, _KT), lambda i: (0, 0, 0)),
            pl.BlockSpec((_M, _S, _K, 3 * _D), lambda i: (0, 0, 0, 0)),
            pl.BlockSpec((_M, 1, _KT), lambda i: (0, 0, 0)),
            pl.BlockSpec((_M, _DT, _DT), lambda i: (0, 0, 0)),
            pl.BlockSpec((_M, 1, _DT), lambda i: (0, 0, 0)),
        ],
        out_specs=[
            pl.BlockSpec((bblk, _S * _M), lambda i: (i, 0)),
            pl.BlockSpec((bblk, _DT), lambda i: (i, 0)),
        ],
        out_shape=[
            jax.ShapeDtypeStruct((B, _S * _M), jnp.int32),
            jax.ShapeDtypeStruct((B, _DT), f32),
        ],
        compiler_params=pltpu.CompilerParams(
            dimension_semantics=("parallel",),
        ),
    )(x, candT, cand, cnorm, wxbd, bf)
    return codes, xhat
